# Initial kernel scaffold; baseline (speedup 1.0000x reference)
import jax
import jax.numpy as jnp
from jax.experimental import pallas as pl


def kernel(kinematics, particle_ids, charges, W, b, pid_table, charge_table):
    raise NotImplementedError("write your pallas kernel here")


# speedup vs baseline: 5.4178x; 5.4178x over previous
import functools

import jax
import jax.numpy as jnp
from jax.experimental import pallas as pl
from jax.experimental.pallas import tpu as pltpu

_B, _N = 4096, 128
_R = _B * _N
_KIN_DIM = 128
_EMB_DIM = 64
_PID_PAD = 32
_CH_PAD = 8
_BR = 4096


def _body(kin_ref, ids_ref, ch_ref, w_ref, b_ref, pidt_ref, cht_ref, out_ref):
    kin = kin_ref[...]
    ids = ids_ref[...]
    ch = ch_ref[...]
    kin_emb = jax.lax.dot_general(
        kin, w_ref[...], (((1,), (0,)), ((), ())),
        preferred_element_type=jnp.float32) + b_ref[...]
    oh_p = (ids[:, None] == jax.lax.broadcasted_iota(
        jnp.int32, (_BR, _PID_PAD), 1)).astype(jnp.float32)
    pid_emb = jax.lax.dot_general(
        oh_p, pidt_ref[...], (((1,), (0,)), ((), ())),
        preferred_element_type=jnp.float32)
    oh_c = ((ch[:, None] + 1) == jax.lax.broadcasted_iota(
        jnp.int32, (_BR, _CH_PAD), 1)).astype(jnp.float32)
    ch_emb = jax.lax.dot_general(
        oh_c, cht_ref[...], (((1,), (0,)), ((), ())),
        preferred_element_type=jnp.float32)
    out_ref[:, 0:_KIN_DIM] = kin_emb
    out_ref[:, _KIN_DIM:_KIN_DIM + _EMB_DIM] = pid_emb
    out_ref[:, _KIN_DIM + _EMB_DIM:] = ch_emb


@functools.partial(jax.jit, static_argnames=("interpret",))
def _run(kinematics, particle_ids, charges, W, b, pid_table, charge_table,
         interpret=False):
    kin = kinematics.reshape(_R, 4)
    ids = particle_ids.reshape(_R)
    ch = charges.reshape(_R)
    b2 = b.reshape(1, _KIN_DIM)
    pidt = jnp.zeros((_PID_PAD, _EMB_DIM), jnp.float32).at[:20].set(pid_table)
    cht = jnp.zeros((_CH_PAD, _EMB_DIM), jnp.float32).at[:3].set(charge_table)
    grid = (_R // _BR,)
    out = pl.pallas_call(
        _body,
        grid=grid,
        in_specs=[
            pl.BlockSpec((_BR, 4), lambda i: (i, 0)),
            pl.BlockSpec((_BR,), lambda i: (i,)),
            pl.BlockSpec((_BR,), lambda i: (i,)),
            pl.BlockSpec((4, _KIN_DIM), lambda i: (0, 0)),
            pl.BlockSpec((1, _KIN_DIM), lambda i: (0, 0)),
            pl.BlockSpec((_PID_PAD, _EMB_DIM), lambda i: (0, 0)),
            pl.BlockSpec((_CH_PAD, _EMB_DIM), lambda i: (0, 0)),
        ],
        out_specs=pl.BlockSpec((_BR, 256), lambda i: (i, 0)),
        out_shape=jax.ShapeDtypeStruct((_R, 256), jnp.float32),
        interpret=interpret,
    )(kin, ids, ch, W, b2, pidt, cht)
    return out.reshape(_B, _N, 256)


def kernel(kinematics, particle_ids, charges, W, b, pid_table, charge_table):
    return _run(kinematics, particle_ids, charges, W, b, pid_table,
                charge_table)


# speedup vs baseline: 5.7163x; 1.0551x over previous
import functools

import jax
import jax.numpy as jnp
from jax.experimental import pallas as pl
from jax.experimental.pallas import tpu as pltpu

_B, _N = 4096, 128
_R = _B * _N
_KIN_DIM = 128
_EMB_DIM = 64
_PID_PAD = 32
_CH_PAD = 8
_BR = 8192


def _body(kin_ref, ids_ref, ch_ref, w_ref, b_ref, pidt_ref, cht_ref, out_ref):
    kin = kin_ref[...]
    ids = ids_ref[...]
    ch = ch_ref[...]
    kin_emb = jax.lax.dot_general(
        kin, w_ref[...], (((1,), (0,)), ((), ())),
        preferred_element_type=jnp.float32) + b_ref[...]
    oh_p = (ids[:, None] == jax.lax.broadcasted_iota(
        jnp.int32, (_BR, _PID_PAD), 1)).astype(jnp.float32)
    pid_emb = jax.lax.dot_general(
        oh_p, pidt_ref[...], (((1,), (0,)), ((), ())),
        preferred_element_type=jnp.float32)
    oh_c = ((ch[:, None] + 1) == jax.lax.broadcasted_iota(
        jnp.int32, (_BR, _CH_PAD), 1)).astype(jnp.float32)
    ch_emb = jax.lax.dot_general(
        oh_c, cht_ref[...], (((1,), (0,)), ((), ())),
        preferred_element_type=jnp.float32)
    out_ref[:, 0:_KIN_DIM] = kin_emb
    out_ref[:, _KIN_DIM:_KIN_DIM + _EMB_DIM] = pid_emb
    out_ref[:, _KIN_DIM + _EMB_DIM:] = ch_emb


@functools.partial(jax.jit, static_argnames=("interpret",))
def _run(kinematics, particle_ids, charges, W, b, pid_table, charge_table,
         interpret=False):
    kin = kinematics.reshape(_R, 4)
    ids = particle_ids.reshape(_R)
    ch = charges.reshape(_R)
    b2 = b.reshape(1, _KIN_DIM)
    pidt = jnp.zeros((_PID_PAD, _EMB_DIM), jnp.float32).at[:20].set(pid_table)
    cht = jnp.zeros((_CH_PAD, _EMB_DIM), jnp.float32).at[:3].set(charge_table)
    grid = (_R // _BR,)
    out = pl.pallas_call(
        _body,
        grid=grid,
        in_specs=[
            pl.BlockSpec((_BR, 4), lambda i: (i, 0)),
            pl.BlockSpec((_BR,), lambda i: (i,)),
            pl.BlockSpec((_BR,), lambda i: (i,)),
            pl.BlockSpec((4, _KIN_DIM), lambda i: (0, 0)),
            pl.BlockSpec((1, _KIN_DIM), lambda i: (0, 0)),
            pl.BlockSpec((_PID_PAD, _EMB_DIM), lambda i: (0, 0)),
            pl.BlockSpec((_CH_PAD, _EMB_DIM), lambda i: (0, 0)),
        ],
        out_specs=pl.BlockSpec((_BR, 256), lambda i: (i, 0)),
        out_shape=jax.ShapeDtypeStruct((_R, 256), jnp.float32),
        compiler_params=pltpu.CompilerParams(
            dimension_semantics=("parallel",)),
        interpret=interpret,
    )(kin, ids, ch, W, b2, pidt, cht)
    return out.reshape(_B, _N, 256)


def kernel(kinematics, particle_ids, charges, W, b, pid_table, charge_table):
    return _run(kinematics, particle_ids, charges, W, b, pid_table,
                charge_table)


# speedup vs baseline: 15.0244x; 2.6283x over previous
import functools

import jax
import jax.numpy as jnp
from jax.experimental import pallas as pl
from jax.experimental.pallas import tpu as pltpu

_B, _N = 4096, 128
_R = _B * _N
_KIN_DIM = 128
_EMB_DIM = 64
_PID_PAD = 32
_CH_PAD = 8
_BB = 32
_BR = _BB * _N


def _body(kin_ref, ids_ref, ch_ref, w_ref, b_ref, pidt_ref, cht_ref, out_ref):
    ids = ids_ref[...]
    ch = ch_ref[...]
    for i in range(_BB):
        kin_emb = jax.lax.dot_general(
            kin_ref[i], w_ref[...], (((0,), (0,)), ((), ())),
            preferred_element_type=jnp.float32)
        out_ref[i * _N:(i + 1) * _N, 0:_KIN_DIM] = kin_emb + b_ref[...]
    oh_p = (ids[:, None] == jax.lax.broadcasted_iota(
        jnp.int32, (_BR, _PID_PAD), 1)).astype(jnp.float32)
    pid_emb = jax.lax.dot_general(
        oh_p, pidt_ref[...], (((1,), (0,)), ((), ())),
        preferred_element_type=jnp.float32)
    oh_c = ((ch[:, None] + 1) == jax.lax.broadcasted_iota(
        jnp.int32, (_BR, _CH_PAD), 1)).astype(jnp.float32)
    ch_emb = jax.lax.dot_general(
        oh_c, cht_ref[...], (((1,), (0,)), ((), ())),
        preferred_element_type=jnp.float32)
    out_ref[:, _KIN_DIM:_KIN_DIM + _EMB_DIM] = pid_emb
    out_ref[:, _KIN_DIM + _EMB_DIM:] = ch_emb


@functools.partial(jax.jit, static_argnames=("interpret",))
def _run(kinematics, particle_ids, charges, W, b, pid_table, charge_table,
         interpret=False):
    kin_t = jnp.transpose(kinematics, (0, 2, 1))
    ids = particle_ids.reshape(_R)
    ch = charges.reshape(_R)
    b2 = b.reshape(1, _KIN_DIM)
    pidt = jnp.zeros((_PID_PAD, _EMB_DIM), jnp.float32).at[:20].set(pid_table)
    cht = jnp.zeros((_CH_PAD, _EMB_DIM), jnp.float32).at[:3].set(charge_table)
    grid = (_B // _BB,)
    out = pl.pallas_call(
        _body,
        grid=grid,
        in_specs=[
            pl.BlockSpec((_BB, 4, _N), lambda i: (i, 0, 0)),
            pl.BlockSpec((_BR,), lambda i: (i,)),
            pl.BlockSpec((_BR,), lambda i: (i,)),
            pl.BlockSpec((4, _KIN_DIM), lambda i: (0, 0)),
            pl.BlockSpec((1, _KIN_DIM), lambda i: (0, 0)),
            pl.BlockSpec((_PID_PAD, _EMB_DIM), lambda i: (0, 0)),
            pl.BlockSpec((_CH_PAD, _EMB_DIM), lambda i: (0, 0)),
        ],
        out_specs=pl.BlockSpec((_BR, 256), lambda i: (i, 0)),
        out_shape=jax.ShapeDtypeStruct((_R, 256), jnp.float32),
        compiler_params=pltpu.CompilerParams(
            dimension_semantics=("parallel",)),
        interpret=interpret,
    )(kin_t, ids, ch, W, b2, pidt, cht)
    return out.reshape(_B, _N, 256)


def kernel(kinematics, particle_ids, charges, W, b, pid_table, charge_table):
    return _run(kinematics, particle_ids, charges, W, b, pid_table,
                charge_table)


# speedup vs baseline: 21.8192x; 1.4523x over previous
import functools

import jax
import jax.numpy as jnp
from jax.experimental import pallas as pl
from jax.experimental.pallas import tpu as pltpu

_B, _N = 4096, 128
_R = _B * _N
_KIN_DIM = 128
_EMB_DIM = 64
_VOC = 32
_BB = 32
_BR = _BB * _N


def _body(kin_ref, ids_ref, ch_ref, w_ref, b_ref, tab_ref, out_ref):
    ids = ids_ref[...]
    ch = ch_ref[...]
    for i in range(_BB):
        kin_emb = jax.lax.dot_general(
            kin_ref[i], w_ref[...], (((0,), (0,)), ((), ())),
            preferred_element_type=jnp.float32)
        out_ref[i * _N:(i + 1) * _N, 0:_KIN_DIM] = kin_emb + b_ref[...]
    rows = jax.lax.broadcasted_iota(jnp.int32, (_VOC, _BR), 0)
    ids_b = jnp.broadcast_to(ids[None, :], (_VOC, _BR))
    chp_b = jnp.broadcast_to((ch + 21)[None, :], (_VOC, _BR))
    oh = (rows == jnp.where(rows < 20, ids_b, chp_b)).astype(jnp.float32)
    emb = jax.lax.dot_general(
        oh, tab_ref[...], (((0,), (0,)), ((), ())),
        preferred_element_type=jnp.float32)
    out_ref[:, _KIN_DIM:] = emb


@functools.partial(jax.jit, static_argnames=("interpret",))
def _run(kinematics, particle_ids, charges, W, b, pid_table, charge_table,
         interpret=False):
    kin_t = jnp.transpose(kinematics, (0, 2, 1))
    ids = particle_ids.reshape(_R)
    ch = charges.reshape(_R)
    b2 = b.reshape(1, _KIN_DIM)
    tab = jnp.zeros((_VOC, 2 * _EMB_DIM), jnp.float32)
    tab = tab.at[:20, :_EMB_DIM].set(pid_table)
    tab = tab.at[20:23, _EMB_DIM:].set(charge_table)
    grid = (_B // _BB,)
    out = pl.pallas_call(
        _body,
        grid=grid,
        in_specs=[
            pl.BlockSpec((_BB, 4, _N), lambda i: (i, 0, 0)),
            pl.BlockSpec((_BR,), lambda i: (i,)),
            pl.BlockSpec((_BR,), lambda i: (i,)),
            pl.BlockSpec((4, _KIN_DIM), lambda i: (0, 0)),
            pl.BlockSpec((1, _KIN_DIM), lambda i: (0, 0)),
            pl.BlockSpec((_VOC, 2 * _EMB_DIM), lambda i: (0, 0)),
        ],
        out_specs=pl.BlockSpec((_BR, 256), lambda i: (i, 0)),
        out_shape=jax.ShapeDtypeStruct((_R, 256), jnp.float32),
        compiler_params=pltpu.CompilerParams(
            dimension_semantics=("parallel",)),
        interpret=interpret,
    )(kin_t, ids, ch, W, b2, tab)
    return out.reshape(_B, _N, 256)


def kernel(kinematics, particle_ids, charges, W, b, pid_table, charge_table):
    return _run(kinematics, particle_ids, charges, W, b, pid_table,
                charge_table)
